# baseline (device time: 52130 ns/iter reference)
import jax
import jax.numpy as jnp
from jax import lax
from jax.experimental import pallas as pl
from jax.experimental.pallas import tpu as pltpu

T = 2048
V_LOCAL = 16384
D = 1024
H = T // 2
CS = 256
NJ = 2


def _vembed(ids, cnt, mask, E):
    def body(ids_ref, cnt_ref, mask_ref, e_ref, out_ref, part, partb,
             recvxb, comp, recvy, recvz, recvyf, recvzf,
             gsems, sx, rx, syo, ryo, szo, rzo, syf, ryf, szf, rzf):
        my_x = lax.axis_index("x")
        my_y = lax.axis_index("y")
        my_z = lax.axis_index("z")
        px = (1 - my_x, my_y, my_z)
        py = (my_x, 1 - my_y, my_z)
        pz = (my_x, my_y, 1 - my_z)
        h0 = my_z * H
        h1 = (1 - my_z) * H

        def g_copy(i, c):
            j = i // CS
            r = lax.rem(i, CS)
            t = h0 + (2 * j + my_y) * CS + r
            local = ids_ref[t] - my_x * V_LOCAL
            safe = jnp.clip(local, 0, V_LOCAL - 1)
            return pltpu.make_async_copy(
                e_ref.at[pl.ds(safe, 1), :],
                part.at[pl.ds(i, 1), :],
                gsems.at[c],
            )

        def issue(i, carry):
            j = i // CS
            r = lax.rem(i, CS)
            t = h0 + (2 * j + my_y) * CS + r
            local = ids_ref[t] - my_x * V_LOCAL

            @pl.when(jnp.logical_and(local >= 0, local < V_LOCAL))
            def _():
                g_copy(i, j).start()

            return carry

        lax.fori_loop(0, NJ * CS, issue, 0, unroll=8)

        bsem = pltpu.get_barrier_semaphore()
        for nbr in (px, py, pz):
            pl.semaphore_signal(bsem, inc=1, device_id=nbr,
                                device_id_type=pl.DeviceIdType.MESH)
        pl.semaphore_wait(bsem, 3)

        def rdma(src, dst, ssem, rsem, dev):
            return pltpu.make_async_remote_copy(
                src_ref=src, dst_ref=dst, send_sem=ssem, recv_sem=rsem,
                device_id=dev, device_id_type=pl.DeviceIdType.MESH)

        def xr(j):
            sl = pl.ds(j * CS, CS)
            return rdma(partb.at[sl], recvxb.at[sl], sx.at[j], rx.at[j], px)

        def yo(j):
            sl = pl.ds(j * CS, CS)
            return rdma(comp.at[sl], recvy.at[sl], syo.at[j], ryo.at[j], py)

        def zo(j):
            sl = pl.ds(j * CS, CS)
            return rdma(comp.at[sl], recvz.at[sl], szo.at[j], rzo.at[j], pz)

        def yf():
            return rdma(recvz.at[pl.ds(0, CS)], recvyf,
                        syf.at[0], ryf.at[0], py)

        def zf():
            return rdma(recvy.at[pl.ds(CS, CS)], recvzf,
                        szf.at[0], rzf.at[0], pz)

        def own_off(j):
            return h0 + (2 * j + my_y) * CS

        def px_process(j):
            xr(j).wait_recv()
            sl = pl.ds(j * CS, CS)
            gsl = pl.ds(own_off(j), CS)
            m = mask_ref[gsl, :]
            cj = jnp.where(m != 0, partb[sl, :], recvxb[sl, :])
            comp[sl, :] = cj
            out_ref[gsl, :] = cj.astype(jnp.float32)
            yo(j).start()
            zo(j).start()

        for j in range(NJ):
            def drain(i, carry, j=j):
                g_copy(0, j).wait()
                return carry

            lax.fori_loop(0, cnt_ref[j], drain, 0)
            sl = pl.ds(j * CS, CS)
            partb[sl, :] = part[sl, :].astype(jnp.bfloat16)
            xr(j).start()
            if j == 1:
                px_process(0)
        px_process(NJ - 1)

        zo(0).wait_recv()
        out_ref[pl.ds(h1 + my_y * CS, CS), :] = (
            recvz[pl.ds(0, CS), :].astype(jnp.float32))
        yf().start()

        yo(0).wait_recv()
        out_ref[pl.ds(h0 + (1 - my_y) * CS, CS), :] = (
            recvy[pl.ds(0, CS), :].astype(jnp.float32))

        zo(1).wait_recv()
        out_ref[pl.ds(h1 + (2 + my_y) * CS, CS), :] = (
            recvz[pl.ds(CS, CS), :].astype(jnp.float32))

        yo(1).wait_recv()
        out_ref[pl.ds(h0 + (2 + 1 - my_y) * CS, CS), :] = (
            recvy[pl.ds(CS, CS), :].astype(jnp.float32))
        zf().start()

        yf().wait_recv()
        out_ref[pl.ds(h1 + (1 - my_y) * CS, CS), :] = (
            recvyf[...].astype(jnp.float32))
        zf().wait_recv()
        out_ref[pl.ds(h1 + (2 + 1 - my_y) * CS, CS), :] = (
            recvzf[...].astype(jnp.float32))

        for j in range(NJ):
            xr(j).wait_send()
            yo(j).wait_send()
            zo(j).wait_send()
        yf().wait_send()
        zf().wait_send()

    return pl.pallas_call(
        body,
        out_shape=jax.ShapeDtypeStruct((T, D), jnp.float32),
        in_specs=[
            pl.BlockSpec(memory_space=pltpu.SMEM),
            pl.BlockSpec(memory_space=pltpu.SMEM),
            pl.BlockSpec(memory_space=pltpu.VMEM),
            pl.BlockSpec(memory_space=pltpu.MemorySpace.HBM),
        ],
        out_specs=pl.BlockSpec(memory_space=pltpu.VMEM),
        scratch_shapes=[
            pltpu.VMEM((NJ * CS, D), jnp.float32),
            pltpu.VMEM((NJ * CS, D), jnp.bfloat16),
            pltpu.VMEM((NJ * CS, D), jnp.bfloat16),
            pltpu.VMEM((NJ * CS, D), jnp.bfloat16),
            pltpu.VMEM((NJ * CS, D), jnp.bfloat16),
            pltpu.VMEM((NJ * CS, D), jnp.bfloat16),
            pltpu.VMEM((CS, D), jnp.bfloat16),
            pltpu.VMEM((CS, D), jnp.bfloat16),
            pltpu.SemaphoreType.DMA((NJ,)),
            pltpu.SemaphoreType.DMA((NJ,)),
            pltpu.SemaphoreType.DMA((NJ,)),
            pltpu.SemaphoreType.DMA((NJ,)),
            pltpu.SemaphoreType.DMA((NJ,)),
            pltpu.SemaphoreType.DMA((NJ,)),
            pltpu.SemaphoreType.DMA((NJ,)),
            pltpu.SemaphoreType.DMA((1,)),
            pltpu.SemaphoreType.DMA((1,)),
            pltpu.SemaphoreType.DMA((1,)),
            pltpu.SemaphoreType.DMA((1,)),
        ],
        compiler_params=pltpu.CompilerParams(collective_id=0),
    )(ids, cnt, mask, E)


def kernel(ids, E):
    my_x = lax.axis_index("x")
    my_y = lax.axis_index("y")
    my_z = lax.axis_index("z")
    local = ids - my_x * V_LOCAL
    in_range = (local >= 0) & (local < V_LOCAL)
    mask = in_range.astype(jnp.float32)[:, None]
    blk_cnt = in_range.astype(jnp.int32).reshape(2 * NJ * 2, CS).sum(axis=1)
    idx = my_z * 4 + 2 * jnp.arange(NJ, dtype=jnp.int32) + my_y
    cnt = jnp.take(blk_cnt, idx, axis=0)
    return _vembed(ids, cnt, mask, E)


# device time: 47460 ns/iter; 1.0984x vs baseline; 1.0984x over previous
import jax
import jax.numpy as jnp
from jax import lax
from jax.experimental import pallas as pl
from jax.experimental.pallas import tpu as pltpu

T = 2048
V_LOCAL = 16384
D = 1024
H = T // 2
CS = 256
NJ = 2


def _vembed(ids, mask, E):
    def body(ids_ref, mask_ref, e_ref, out_ref, part, partb, recvxb,
             comp, recvy, recvz, recvyf, recvzf,
             gsems, sx, rx, syo, ryo, szo, rzo, syf, ryf, szf, rzf):
        my_x = lax.axis_index("x")
        my_y = lax.axis_index("y")
        my_z = lax.axis_index("z")
        px = (1 - my_x, my_y, my_z)
        py = (my_x, 1 - my_y, my_z)
        pz = (my_x, my_y, 1 - my_z)
        h0 = my_z * H
        h1 = (1 - my_z) * H

        def g_copy(i, c):
            j = i // CS
            r = lax.rem(i, CS)
            t = h0 + (2 * j + my_y) * CS + r
            local = ids_ref[t] - my_x * V_LOCAL
            safe = jnp.clip(local, 0, V_LOCAL - 1)
            return pltpu.make_async_copy(
                e_ref.at[pl.ds(safe, 1), :],
                part.at[pl.ds(i, 1), :],
                gsems.at[c],
            )

        def issue(i, carry):
            g_copy(i, i // CS).start()
            return carry

        lax.fori_loop(0, NJ * CS, issue, 0, unroll=8)

        bsem = pltpu.get_barrier_semaphore()
        for nbr in (px, py, pz):
            pl.semaphore_signal(bsem, inc=1, device_id=nbr,
                                device_id_type=pl.DeviceIdType.MESH)
        pl.semaphore_wait(bsem, 3)

        def rdma(src, dst, ssem, rsem, dev):
            return pltpu.make_async_remote_copy(
                src_ref=src, dst_ref=dst, send_sem=ssem, recv_sem=rsem,
                device_id=dev, device_id_type=pl.DeviceIdType.MESH)

        def xr(j):
            sl = pl.ds(j * CS, CS)
            return rdma(partb.at[sl], recvxb.at[sl], sx.at[j], rx.at[j], px)

        def yo(j):
            sl = pl.ds(j * CS, CS)
            return rdma(comp.at[sl], recvy.at[sl], syo.at[j], ryo.at[j], py)

        def zo(j):
            sl = pl.ds(j * CS, CS)
            return rdma(comp.at[sl], recvz.at[sl], szo.at[j], rzo.at[j], pz)

        def yf():
            return rdma(recvz.at[pl.ds(0, CS)], recvyf,
                        syf.at[0], ryf.at[0], py)

        def zf():
            return rdma(recvy.at[pl.ds(CS, CS)], recvzf,
                        szf.at[0], rzf.at[0], pz)

        def own_off(j):
            return h0 + (2 * j + my_y) * CS

        def px_process(j):
            xr(j).wait_recv()
            sl = pl.ds(j * CS, CS)
            gsl = pl.ds(own_off(j), CS)
            m = mask_ref[gsl, :]
            cj = jnp.where(m != 0, partb[sl, :], recvxb[sl, :])
            comp[sl, :] = cj
            out_ref[gsl, :] = cj
            yo(j).start()
            zo(j).start()

        for j in range(NJ):
            def drain(i, carry, j=j):
                g_copy(0, j).wait()
                return carry

            lax.fori_loop(0, CS, drain, 0, unroll=8)
            sl = pl.ds(j * CS, CS)
            partb[sl, :] = part[sl, :].astype(jnp.bfloat16)
            xr(j).start()
            if j == 1:
                px_process(0)
        px_process(NJ - 1)

        zo(0).wait_recv()
        out_ref[pl.ds(h1 + my_y * CS, CS), :] = recvz[pl.ds(0, CS), :]
        yf().start()

        yo(0).wait_recv()
        out_ref[pl.ds(h0 + (1 - my_y) * CS, CS), :] = recvy[pl.ds(0, CS), :]

        zo(1).wait_recv()
        out_ref[pl.ds(h1 + (2 + my_y) * CS, CS), :] = recvz[pl.ds(CS, CS), :]

        yo(1).wait_recv()
        out_ref[pl.ds(h0 + (2 + 1 - my_y) * CS, CS), :] = recvy[pl.ds(CS, CS), :]
        zf().start()

        yf().wait_recv()
        out_ref[pl.ds(h1 + (1 - my_y) * CS, CS), :] = recvyf[...]
        zf().wait_recv()
        out_ref[pl.ds(h1 + (2 + 1 - my_y) * CS, CS), :] = recvzf[...]

        for j in range(NJ):
            xr(j).wait_send()
            yo(j).wait_send()
            zo(j).wait_send()
        yf().wait_send()
        zf().wait_send()

    return pl.pallas_call(
        body,
        out_shape=jax.ShapeDtypeStruct((T, D), jnp.bfloat16),
        in_specs=[
            pl.BlockSpec(memory_space=pltpu.SMEM),
            pl.BlockSpec(memory_space=pltpu.VMEM),
            pl.BlockSpec(memory_space=pltpu.MemorySpace.HBM),
        ],
        out_specs=pl.BlockSpec(memory_space=pltpu.VMEM),
        scratch_shapes=[
            pltpu.VMEM((NJ * CS, D), jnp.float32),
            pltpu.VMEM((NJ * CS, D), jnp.bfloat16),
            pltpu.VMEM((NJ * CS, D), jnp.bfloat16),
            pltpu.VMEM((NJ * CS, D), jnp.bfloat16),
            pltpu.VMEM((NJ * CS, D), jnp.bfloat16),
            pltpu.VMEM((NJ * CS, D), jnp.bfloat16),
            pltpu.VMEM((CS, D), jnp.bfloat16),
            pltpu.VMEM((CS, D), jnp.bfloat16),
            pltpu.SemaphoreType.DMA((NJ,)),
            pltpu.SemaphoreType.DMA((NJ,)),
            pltpu.SemaphoreType.DMA((NJ,)),
            pltpu.SemaphoreType.DMA((NJ,)),
            pltpu.SemaphoreType.DMA((NJ,)),
            pltpu.SemaphoreType.DMA((NJ,)),
            pltpu.SemaphoreType.DMA((NJ,)),
            pltpu.SemaphoreType.DMA((1,)),
            pltpu.SemaphoreType.DMA((1,)),
            pltpu.SemaphoreType.DMA((1,)),
            pltpu.SemaphoreType.DMA((1,)),
        ],
        compiler_params=pltpu.CompilerParams(collective_id=0),
    )(ids, mask, E)


def kernel(ids, E):
    my_x = lax.axis_index("x")
    local = ids - my_x * V_LOCAL
    in_range = (local >= 0) & (local < V_LOCAL)
    mask = in_range.astype(jnp.float32)[:, None]
    return _vembed(ids, mask, E)
